# baseline (speedup 1.0000x reference)
import jax
import jax.numpy as jnp
from jax.experimental import pallas as pl
from jax.experimental.pallas import tpu as pltpu

_H = 512
_W = 512
_N = _H * _W
_B = 8
_TOPK = 30
_G = 64
_GR = _H // _G
_NEG = -1.0e9
_BIGI = 1 << 24


def _body(pred_ref, targ_ref, outD_ref, outR_ref, *scratch):
    a_refs = scratch[0:_B]
    b_refs = scratch[_B:2 * _B]
    gma_sc, gmb_sc = scratch[2 * _B], scratch[2 * _B + 1]
    b = pl.program_id(0)

    p0 = pred_ref[0, 0]
    p1 = pred_ref[0, 1]
    t = targ_ref[0]

    s = 1.0 / (1.0 + jnp.exp(p0 - p1))
    st = s * t
    s1 = jnp.sum(s)
    ts = jnp.sum(t)
    iv = jnp.sum(st)

    smooth = 1e-5
    n = float(_N)
    dice1 = 1.0 - 2.0 * iv / (s1 + ts + smooth)
    i0 = n - s1 - ts + iv
    dice0 = 1.0 - 2.0 * i0 / (2.0 * n - s1 - ts + smooth)

    a3 = (s - st).reshape(_G, _GR, _W)
    b3 = (t - st).reshape(_G, _GR, _W)
    for i in range(_B):
        @pl.when(b == i)
        def _(i=i):
            a_refs[i][...] = a3
            b_refs[i][...] = b3
    gma_sc[pl.ds(b, 1), :] = jnp.max(a3, axis=(1, 2)).reshape(1, _G)
    gmb_sc[pl.ds(b, 1), :] = jnp.max(b3, axis=(1, 2)).reshape(1, _G)

    @pl.when(b == 0)
    def _():
        outD_ref[...] = jnp.zeros((1, 1), jnp.float32)

    outD_ref[...] += jnp.full((1, 1), (dice0 + dice1) / (2.0 * _B))

    @pl.when(b == _B - 1)
    def _():
        i64r = jax.lax.broadcasted_iota(jnp.int32, (_B, _G), 1)
        r8g = jax.lax.broadcasted_iota(jnp.int32, (_B, _G), 0)
        l128 = jax.lax.broadcasted_iota(jnp.int32, (_B, 128), 1)
        r8l = jax.lax.broadcasted_iota(jnp.int32, (_B, 128), 0)
        fi = (jax.lax.broadcasted_iota(jnp.int32, (1, _GR, _W), 1) * _W
              + jax.lax.broadcasted_iota(jnp.int32, (1, _GR, _W), 2))

        def visit(gm, acc, refs, bb, gi8, slot, k):
            gi = gi8[bb, 0]
            grp = refs[bb][pl.ds(gi, 1)]
            m1 = jnp.max(grp, axis=(1, 2), keepdims=True)
            l1 = jnp.min(jnp.where(grp == jnp.broadcast_to(m1, grp.shape),
                                   fi, _BIGI), axis=(1, 2), keepdims=True)
            grp = jnp.where(fi == jnp.broadcast_to(l1, fi.shape), _NEG, grp)
            m2 = jnp.max(grp, axis=(1, 2), keepdims=True)
            l2 = jnp.min(jnp.where(grp == jnp.broadcast_to(m2, grp.shape),
                                   fi, _BIGI), axis=(1, 2), keepdims=True)
            grp = jnp.where(fi == jnp.broadcast_to(l2, fi.shape), _NEG, grp)
            refs[bb][pl.ds(gi, 1)] = grp
            nm = jnp.max(grp, axis=(1, 2), keepdims=True)
            gm = jnp.where((r8g == bb) & (i64r == gi),
                           jnp.broadcast_to(nm.reshape(1, 1), (_B, _G)), gm)
            acc = jnp.where((r8l == bb) & (l128 == 4 * k + slot),
                            jnp.broadcast_to(m1.reshape(1, 1), (_B, 128)), acc)
            acc = jnp.where((r8l == bb) & (l128 == 4 * k + slot + 1),
                            jnp.broadcast_to(m2.reshape(1, 1), (_B, 128)), acc)
            return gm, acc

        def extract_all(k, gm, refs, acc):
            g8a = jnp.max(gm, axis=1, keepdims=True)
            gi8a = jnp.min(jnp.where(gm == g8a, i64r, _G), axis=1,
                           keepdims=True)
            gmx = jnp.where(i64r == gi8a, _NEG, gm)
            g8b = jnp.max(gmx, axis=1, keepdims=True)
            gi8b = jnp.min(jnp.where(gmx == g8b, i64r, _G), axis=1,
                           keepdims=True)
            for bb in range(_B):
                gm, acc = visit(gm, acc, refs, bb, gi8a, 0, k)
                gm, acc = visit(gm, acc, refs, bb, gi8b, 2, k)
            return gm, acc

        def step(k, carry):
            gma, gmb, acca, accb = carry
            gma, acca = extract_all(k, gma, a_refs, acca)
            gmb, accb = extract_all(k, gmb, b_refs, accb)
            return gma, gmb, acca, accb

        init = (gma_sc[...], gmb_sc[...],
                jnp.full((_B, 128), _NEG, jnp.float32),
                jnp.full((_B, 128), _NEG, jnp.float32))
        _, _, acca, accb = jax.lax.fori_loop(0, _TOPK // 2, step, init)

        def select_top(acc):
            top = jnp.full((_B, 128), _NEG, jnp.float32)
            for j in range(_TOPK):
                m = jnp.max(acc, axis=1, keepdims=True)
                li = jnp.min(jnp.where(acc == m, l128, 128), axis=1,
                             keepdims=True)
                acc = jnp.where(l128 == li, _NEG, acc)
                top = jnp.where(l128 == j, jnp.broadcast_to(m, (_B, 128)),
                                top)
            return top

        acca = select_top(acca)
        accb = select_top(accb)
        hinge = jnp.zeros((_B, 128), jnp.float32)
        for i in range(_TOPK):
            th = acca[:, i:i + 1] + accb - 0.7
            hinge = hinge + jnp.maximum(th, 0.0)
        outR_ref[...] = jnp.full((1, 1),
                                 jnp.sum(hinge) / (_B * _TOPK * _TOPK))


def kernel(predicts, target):
    outD, outR = pl.pallas_call(
        _body,
        grid=(_B,),
        in_specs=[
            pl.BlockSpec((1, 2, _H, _W), lambda b: (b, 0, 0, 0)),
            pl.BlockSpec((1, _H, _W), lambda b: (b, 0, 0)),
        ],
        out_specs=[
            pl.BlockSpec((1, 1), lambda b: (0, 0)),
            pl.BlockSpec((1, 1), lambda b: (0, 0)),
        ],
        out_shape=[
            jax.ShapeDtypeStruct((1, 1), jnp.float32),
            jax.ShapeDtypeStruct((1, 1), jnp.float32),
        ],
        scratch_shapes=(
            [pltpu.VMEM((_G, _GR, _W), jnp.float32) for _ in range(2 * _B)]
            + [pltpu.VMEM((_B, _G), jnp.float32),
               pltpu.VMEM((_B, _G), jnp.float32)]
        ),
        compiler_params=pltpu.CompilerParams(
            dimension_semantics=("arbitrary",),
        ),
    )(predicts, target)
    return (outD[0, 0], outR[0, 0])


# speedup vs baseline: 1.4593x; 1.4593x over previous
import jax
import jax.numpy as jnp
from jax.experimental import pallas as pl
from jax.experimental.pallas import tpu as pltpu

_H = 512
_W = 512
_N = _H * _W
_B = 8
_TOPK = 30
_G = 64
_GR = _H // _G
_NEG = -1.0e9
_BIGI = 1 << 24


def _body(pred_ref, targ_ref, outD_ref, outR_ref, *scratch):
    a_refs = scratch[0:_B]
    b_refs = scratch[_B:2 * _B]
    gma_sc, gmb_sc = scratch[2 * _B], scratch[2 * _B + 1]
    b = pl.program_id(0)

    p0 = pred_ref[0, 0]
    p1 = pred_ref[0, 1]
    t = targ_ref[0]

    s = 1.0 / (1.0 + jnp.exp(p0 - p1))
    st = s * t
    s1 = jnp.sum(s)
    ts = jnp.sum(t)
    iv = jnp.sum(st)

    smooth = 1e-5
    n = float(_N)
    dice1 = 1.0 - 2.0 * iv / (s1 + ts + smooth)
    i0 = n - s1 - ts + iv
    dice0 = 1.0 - 2.0 * i0 / (2.0 * n - s1 - ts + smooth)

    a3 = (s - st).reshape(_G, _GR, _W)
    b3 = (t - st).reshape(_G, _GR, _W)
    for i in range(_B):
        @pl.when(b == i)
        def _(i=i):
            a_refs[i][...] = a3
            b_refs[i][...] = b3
    gma_sc[pl.ds(b, 1), :] = jnp.max(a3, axis=(1, 2)).reshape(1, _G)
    gmb_sc[pl.ds(b, 1), :] = jnp.max(b3, axis=(1, 2)).reshape(1, _G)

    @pl.when(b == 0)
    def _():
        outD_ref[...] = jnp.zeros((1, 1), jnp.float32)

    outD_ref[...] += jnp.full((1, 1), (dice0 + dice1) / (2.0 * _B))

    @pl.when(b == _B - 1)
    def _():
        i64r = jax.lax.broadcasted_iota(jnp.int32, (_B, _G), 1)
        r8g = jax.lax.broadcasted_iota(jnp.int32, (_B, _G), 0)
        l128 = jax.lax.broadcasted_iota(jnp.int32, (_B, 128), 1)
        r8l = jax.lax.broadcasted_iota(jnp.int32, (_B, 128), 0)
        fi = (jax.lax.broadcasted_iota(jnp.int32, (1, _GR, _W), 1) * _W
              + jax.lax.broadcasted_iota(jnp.int32, (1, _GR, _W), 2))

        def extract_all(k, gm, refs, acc):
            g8 = jnp.max(gm, axis=1, keepdims=True)
            gi8 = jnp.min(jnp.where(gm == g8, i64r, _G), axis=1,
                          keepdims=True)
            for bb in range(_B):
                gi = gi8[bb, 0]
                gv = g8[bb:bb + 1, 0:1].reshape(1, 1, 1)
                grp = refs[bb][pl.ds(gi, 1)]
                mask = grp == jnp.broadcast_to(gv, grp.shape)
                locv = jnp.min(jnp.where(mask, fi, _BIGI), axis=(1, 2),
                               keepdims=True)
                grp = jnp.where(fi == jnp.broadcast_to(locv, fi.shape),
                                _NEG, grp)
                refs[bb][pl.ds(gi, 1)] = grp
                nmv = jnp.max(grp, axis=(1, 2), keepdims=True)
                nm64 = jnp.broadcast_to(nmv.reshape(1, 1), (_B, _G))
                gm = jnp.where((r8g == bb) & (i64r == gi), nm64, gm)
                gacc = jnp.broadcast_to(g8[bb:bb + 1, 0:1], (_B, 128))
                acc = jnp.where((r8l == bb) & (l128 == k), gacc, acc)
            return gm, acc

        def step(k, carry):
            gma, gmb, acca, accb = carry
            gma, acca = extract_all(k, gma, a_refs, acca)
            gmb, accb = extract_all(k, gmb, b_refs, accb)
            return gma, gmb, acca, accb

        init = (gma_sc[...], gmb_sc[...],
                jnp.full((_B, 128), _NEG, jnp.float32),
                jnp.full((_B, 128), _NEG, jnp.float32))
        _, _, acca, accb = jax.lax.fori_loop(0, _TOPK, step, init)
        hinge = jnp.zeros((_B, 128), jnp.float32)
        for i in range(_TOPK):
            th = acca[:, i:i + 1] + accb - 0.7
            hinge = hinge + jnp.maximum(th, 0.0)
        outR_ref[...] = jnp.full((1, 1),
                                 jnp.sum(hinge) / (_B * _TOPK * _TOPK))


def kernel(predicts, target):
    outD, outR = pl.pallas_call(
        _body,
        grid=(_B,),
        in_specs=[
            pl.BlockSpec((1, 2, _H, _W), lambda b: (b, 0, 0, 0)),
            pl.BlockSpec((1, _H, _W), lambda b: (b, 0, 0)),
        ],
        out_specs=[
            pl.BlockSpec((1, 1), lambda b: (0, 0)),
            pl.BlockSpec((1, 1), lambda b: (0, 0)),
        ],
        out_shape=[
            jax.ShapeDtypeStruct((1, 1), jnp.float32),
            jax.ShapeDtypeStruct((1, 1), jnp.float32),
        ],
        scratch_shapes=(
            [pltpu.VMEM((_G, _GR, _W), jnp.float32) for _ in range(2 * _B)]
            + [pltpu.VMEM((_B, _G), jnp.float32),
               pltpu.VMEM((_B, _G), jnp.float32)]
        ),
        compiler_params=pltpu.CompilerParams(
            dimension_semantics=("arbitrary",),
        ),
    )(predicts, target)
    return (outD[0, 0], outR[0, 0])


# speedup vs baseline: 1.4623x; 1.0021x over previous
import jax
import jax.numpy as jnp
from jax.experimental import pallas as pl
from jax.experimental.pallas import tpu as pltpu

_H = 512
_W = 512
_N = _H * _W
_B = 8
_TOPK = 30
_G = 64
_GR = _H // _G
_NEG = -1.0e9
_BIGI = 1 << 24


def _body(pred_ref, targ_ref, outD_ref, outR_ref, *scratch):
    a_refs = scratch[0:_B]
    b_refs = scratch[_B:2 * _B]
    gma_sc, gmb_sc = scratch[2 * _B], scratch[2 * _B + 1]
    b = pl.program_id(0)

    p0 = pred_ref[0, 0]
    p1 = pred_ref[0, 1]
    t = targ_ref[0]

    s = 1.0 / (1.0 + jnp.exp(p0 - p1))
    st = s * t
    s1 = jnp.sum(s)
    ts = jnp.sum(t)
    iv = jnp.sum(st)

    smooth = 1e-5
    n = float(_N)
    dice1 = 1.0 - 2.0 * iv / (s1 + ts + smooth)
    i0 = n - s1 - ts + iv
    dice0 = 1.0 - 2.0 * i0 / (2.0 * n - s1 - ts + smooth)

    a3 = (s - st).reshape(_G, _GR, _W)
    b3 = (t - st).reshape(_G, _GR, _W)
    for i in range(_B):
        @pl.when(b == i)
        def _(i=i):
            a_refs[i][...] = a3
            b_refs[i][...] = b3
    gma_sc[pl.ds(b, 1), :] = jnp.max(a3, axis=(1, 2)).reshape(1, _G)
    gmb_sc[pl.ds(b, 1), :] = jnp.max(b3, axis=(1, 2)).reshape(1, _G)

    @pl.when(b == 0)
    def _():
        outD_ref[...] = jnp.zeros((1, 1), jnp.float32)

    outD_ref[...] += jnp.full((1, 1), (dice0 + dice1) / (2.0 * _B))

    @pl.when(b == _B - 1)
    def _():
        i64r = jax.lax.broadcasted_iota(jnp.int32, (_B, _G), 1)
        r8g = jax.lax.broadcasted_iota(jnp.int32, (_B, _G), 0)
        l128 = jax.lax.broadcasted_iota(jnp.int32, (_B, 128), 1)
        r8l = jax.lax.broadcasted_iota(jnp.int32, (_B, 128), 0)
        fi = (jax.lax.broadcasted_iota(jnp.int32, (1, _GR, _W), 1) * _W
              + jax.lax.broadcasted_iota(jnp.int32, (1, _GR, _W), 2))

        def extract_all(k, gm, refs, acc):
            g8 = jnp.max(gm, axis=1, keepdims=True)
            gi8 = jnp.min(jnp.where(gm == g8, i64r, _G), axis=1,
                          keepdims=True)
            for bb in range(_B):
                gi = gi8[bb, 0]
                gv = g8[bb:bb + 1, 0:1].reshape(1, 1, 1)
                grp = refs[bb][pl.ds(gi, 1)]
                mask = grp == jnp.broadcast_to(gv, grp.shape)
                locv = jnp.min(jnp.where(mask, fi, _BIGI), axis=(1, 2),
                               keepdims=True)
                grp = jnp.where(fi == jnp.broadcast_to(locv, fi.shape),
                                _NEG, grp)
                refs[bb][pl.ds(gi, 1)] = grp
                nmv = jnp.max(grp, axis=(1, 2), keepdims=True)
                nm64 = jnp.broadcast_to(nmv.reshape(1, 1), (_B, _G))
                gm = jnp.where((r8g == bb) & (i64r == gi), nm64, gm)
                gacc = jnp.broadcast_to(g8[bb:bb + 1, 0:1], (_B, 128))
                acc = jnp.where((r8l == bb) & (l128 == k), gacc, acc)
            return gm, acc

        def front(gm):
            g8 = jnp.max(gm, axis=1, keepdims=True)
            gi8 = jnp.min(jnp.where(gm == g8, i64r, _G), axis=1,
                          keepdims=True)
            return g8, gi8

        def unit(k, gm, refs, acc, bb, g8, gi8):
            gi = gi8[bb, 0]
            gv = g8[bb:bb + 1, 0:1].reshape(1, 1, 1)
            grp = refs[bb][pl.ds(gi, 1)]
            mask = grp == jnp.broadcast_to(gv, grp.shape)
            locv = jnp.min(jnp.where(mask, fi, _BIGI), axis=(1, 2),
                           keepdims=True)
            grp = jnp.where(fi == jnp.broadcast_to(locv, fi.shape),
                            _NEG, grp)
            refs[bb][pl.ds(gi, 1)] = grp
            nmv = jnp.max(grp, axis=(1, 2), keepdims=True)
            nm64 = jnp.broadcast_to(nmv.reshape(1, 1), (_B, _G))
            gm = jnp.where((r8g == bb) & (i64r == gi), nm64, gm)
            gacc = jnp.broadcast_to(g8[bb:bb + 1, 0:1], (_B, 128))
            acc = jnp.where((r8l == bb) & (l128 == k), gacc, acc)
            return gm, acc

        def step(k, carry):
            gma, gmb, acca, accb = carry
            g8a, gi8a = front(gma)
            g8b, gi8b = front(gmb)
            for bb in range(_B):
                gma, acca = unit(k, gma, a_refs, acca, bb, g8a, gi8a)
                gmb, accb = unit(k, gmb, b_refs, accb, bb, g8b, gi8b)
            return gma, gmb, acca, accb

        init = (gma_sc[...], gmb_sc[...],
                jnp.full((_B, 128), _NEG, jnp.float32),
                jnp.full((_B, 128), _NEG, jnp.float32))
        _, _, acca, accb = jax.lax.fori_loop(0, _TOPK, step, init)
        hinge = jnp.zeros((_B, 128), jnp.float32)
        for i in range(_TOPK):
            th = acca[:, i:i + 1] + accb - 0.7
            hinge = hinge + jnp.maximum(th, 0.0)
        outR_ref[...] = jnp.full((1, 1),
                                 jnp.sum(hinge) / (_B * _TOPK * _TOPK))


def kernel(predicts, target):
    outD, outR = pl.pallas_call(
        _body,
        grid=(_B,),
        in_specs=[
            pl.BlockSpec((1, 2, _H, _W), lambda b: (b, 0, 0, 0)),
            pl.BlockSpec((1, _H, _W), lambda b: (b, 0, 0)),
        ],
        out_specs=[
            pl.BlockSpec((1, 1), lambda b: (0, 0)),
            pl.BlockSpec((1, 1), lambda b: (0, 0)),
        ],
        out_shape=[
            jax.ShapeDtypeStruct((1, 1), jnp.float32),
            jax.ShapeDtypeStruct((1, 1), jnp.float32),
        ],
        scratch_shapes=(
            [pltpu.VMEM((_G, _GR, _W), jnp.float32) for _ in range(2 * _B)]
            + [pltpu.VMEM((_B, _G), jnp.float32),
               pltpu.VMEM((_B, _G), jnp.float32)]
        ),
        compiler_params=pltpu.CompilerParams(
            dimension_semantics=("arbitrary",),
        ),
    )(predicts, target)
    return (outD[0, 0], outR[0, 0])
